# Initial kernel scaffold; baseline (speedup 1.0000x reference)
import jax
import jax.numpy as jnp
from jax.experimental import pallas as pl


def kernel(x, pos, table):
    raise NotImplementedError("write your pallas kernel here")


# speedup vs baseline: 1.6928x; 1.6928x over previous
import functools

import jax
import jax.numpy as jnp
from jax import lax
from jax.experimental import pallas as pl
from jax.experimental.pallas import tpu as pltpu
from jax.experimental.pallas import tpu_sc as plsc

VOCAB = 1000000
EMB = 64
MAXLEN = 200

NC = 2
NS = 16
NW = NC * NS
LANES = 16
CH = 128


def _sincos_position_encoding(max_length, embedding_dim, n=10000):
    k = jnp.arange(max_length, dtype=jnp.float32)[:, None]
    i = jnp.arange(embedding_dim // 2, dtype=jnp.float32)[None, :]
    denominator = jnp.power(float(n), 2.0 * i / embedding_dim)
    P = jnp.zeros((max_length, embedding_dim), dtype=jnp.float32)
    P = P.at[:, 0::2].set(jnp.sin(k / denominator))
    P = P.at[:, 1::2].set(jnp.cos(k / denominator))
    return P


@functools.partial(jax.jit, static_argnames=("n",))
def _sc_lookup(xf, pf, table, penc, n):
    per_w = n // NW
    nsteps = per_w // CH
    mesh = plsc.VectorSubcoreMesh(core_axis_name="c", subcore_axis_name="s")

    @functools.partial(
        pl.kernel,
        mesh=mesh,
        out_type=jax.ShapeDtypeStruct((n, EMB), jnp.float32),
        scratch_types=[
            pltpu.VMEM((CH,), jnp.int32),
            pltpu.VMEM((CH,), jnp.int32),
            pltpu.VMEM((CH, EMB), jnp.float32),
            pltpu.VMEM((CH, EMB), jnp.float32),
            pltpu.SemaphoreType.DMA,
            pltpu.SemaphoreType.DMA,
        ],
        compiler_params=pltpu.CompilerParams(use_tc_tiling_on_sc=False),
    )
    def k(x_hbm, p_hbm, table_hbm, penc_hbm, out_hbm,
          xi_v, pi_v, rows_v, prow_v, sem_t, sem_p):
        wid = lax.axis_index("s") * NC + lax.axis_index("c")
        w_base = wid * per_w

        def step(i, carry):
            base = w_base + i * CH
            pltpu.sync_copy(x_hbm.at[pl.ds(base, CH)], xi_v)
            pltpu.sync_copy(p_hbm.at[pl.ds(base, CH)], pi_v)
            cp_t = pltpu.async_copy(table_hbm.at[xi_v], rows_v, sem_t)
            cp_p = pltpu.async_copy(penc_hbm.at[pi_v], prow_v, sem_p)
            cp_t.wait()
            cp_p.wait()

            def addrow(r, c):
                for e in range(EMB // LANES):
                    sl = pl.ds(e * LANES, LANES)
                    rows_v[r, sl] = rows_v[r, sl] + prow_v[r, sl]
                return c

            lax.fori_loop(0, CH, addrow, 0, unroll=4)
            pltpu.sync_copy(rows_v, out_hbm.at[pl.ds(base, CH)])
            return carry

        lax.fori_loop(0, nsteps, step, 0)

    return k(xf, pf, table, penc)


def kernel(x, pos, table):
    B, L = x.shape
    n = B * L
    xf = x.reshape(n).astype(jnp.int32)
    pf = pos.reshape(n).astype(jnp.int32)
    penc = _sincos_position_encoding(MAXLEN, EMB)
    out = _sc_lookup(xf, pf, table, penc, n)
    return out.reshape(B, L, EMB)


# speedup vs baseline: 1.9691x; 1.1632x over previous
import functools

import jax
import jax.numpy as jnp
from jax import lax
from jax.experimental import pallas as pl
from jax.experimental.pallas import tpu as pltpu
from jax.experimental.pallas import tpu_sc as plsc

VOCAB = 1000000
EMB = 64
MAXLEN = 200

NC = 2
NS = 16
NW = NC * NS
LANES = 16
CH = 128
NBUF = 4


def _sincos_position_encoding(max_length, embedding_dim, n=10000):
    k = jnp.arange(max_length, dtype=jnp.float32)[:, None]
    i = jnp.arange(embedding_dim // 2, dtype=jnp.float32)[None, :]
    denominator = jnp.power(float(n), 2.0 * i / embedding_dim)
    P = jnp.zeros((max_length, embedding_dim), dtype=jnp.float32)
    P = P.at[:, 0::2].set(jnp.sin(k / denominator))
    P = P.at[:, 1::2].set(jnp.cos(k / denominator))
    return P


@functools.partial(jax.jit, static_argnames=("n",))
def _sc_lookup(xf, pf, table, penc, n):
    per_w = n // NW
    nsteps = per_w // CH
    ngroups = nsteps // NBUF
    mesh = plsc.VectorSubcoreMesh(core_axis_name="c", subcore_axis_name="s")

    @functools.partial(
        pl.kernel,
        mesh=mesh,
        out_type=jax.ShapeDtypeStruct((n, EMB), jnp.float32),
        scratch_types=[
            pltpu.VMEM((per_w,), jnp.int32),
            pltpu.VMEM((per_w,), jnp.int32),
            pltpu.VMEM((NBUF, CH, EMB), jnp.float32),
            pltpu.VMEM((NBUF, CH, EMB), jnp.float32),
            pltpu.SemaphoreType.DMA((NBUF,)),
            pltpu.SemaphoreType.DMA((NBUF,)),
            pltpu.SemaphoreType.DMA((NBUF,)),
        ],
        compiler_params=pltpu.CompilerParams(use_tc_tiling_on_sc=False),
    )
    def k(x_hbm, p_hbm, table_hbm, penc_hbm, out_hbm,
          xi_v, pi_v, rows_v, prow_v, sem_t, sem_p, sem_s):
        cid = lax.axis_index("c")
        sid = lax.axis_index("s")
        wid = sid * NC + cid
        w_base = wid * per_w

        pltpu.sync_copy(x_hbm.at[pl.ds(w_base, per_w)], xi_v)
        pltpu.sync_copy(p_hbm.at[pl.ds(w_base, per_w)], pi_v)

        def issue_gathers(g, b):
            xi = xi_v.at[pl.ds(g * CH, CH)]
            pi = pi_v.at[pl.ds(g * CH, CH)]
            pltpu.async_copy(table_hbm.at[xi], rows_v.at[b], sem_t.at[b])
            pltpu.async_copy(penc_hbm.at[pi], prow_v.at[b], sem_p.at[b])

        def wait_gathers(g, b):
            xi = xi_v.at[pl.ds(g * CH, CH)]
            pi = pi_v.at[pl.ds(g * CH, CH)]
            pltpu.make_async_copy(table_hbm.at[xi], rows_v.at[b], sem_t.at[b]).wait()
            pltpu.make_async_copy(penc_hbm.at[pi], prow_v.at[b], sem_p.at[b]).wait()

        def wait_store(b):
            pltpu.make_async_copy(
                rows_v.at[b], out_hbm.at[pl.ds(w_base, CH)], sem_s.at[b]).wait()

        for b in range(NBUF - 1):
            issue_gathers(b, b)

        def group(gr, carry):
            for b in range(NBUF):
                g = gr * NBUF + b
                pb = (b + NBUF - 1) % NBUF
                pg = g + NBUF - 1
                if b == 0:
                    @pl.when(gr >= 1)
                    def _():
                        wait_store(pb)

                    issue_gathers(pg, pb)
                else:
                    @pl.when(pg < nsteps)
                    def _():
                        wait_store(pb)
                        issue_gathers(pg, pb)

                wait_gathers(g, b)

                def addrow(r, c):
                    for e in range(EMB // LANES):
                        sl = pl.ds(e * LANES, LANES)
                        rows_v[b, r, sl] = rows_v[b, r, sl] + prow_v[b, r, sl]
                    return c

                lax.fori_loop(0, CH, addrow, 0, unroll=4)
                pltpu.async_copy(
                    rows_v.at[b], out_hbm.at[pl.ds(w_base + g * CH, CH)],
                    sem_s.at[b])
            return carry

        lax.fori_loop(0, ngroups, group, 0)

        for j in range(NBUF):
            wait_store((nsteps - NBUF + j) % NBUF)

    return k(xf, pf, table, penc)


def kernel(x, pos, table):
    B, L = x.shape
    n = B * L
    xf = x.reshape(n).astype(jnp.int32)
    pf = pos.reshape(n).astype(jnp.int32)
    penc = _sincos_position_encoding(MAXLEN, EMB)
    out = _sc_lookup(xf, pf, table, penc, n)
    return out.reshape(B, L, EMB)


# speedup vs baseline: 2.1487x; 1.0912x over previous
import functools

import jax
import jax.numpy as jnp
from jax import lax
from jax.experimental import pallas as pl
from jax.experimental.pallas import tpu as pltpu
from jax.experimental.pallas import tpu_sc as plsc

VOCAB = 1000000
EMB = 64
MAXLEN = 200

NC = 2
NS = 16
NW = NC * NS
LANES = 16
CH = 128
NBUF = 4


def _sincos_position_encoding(max_length, embedding_dim, n=10000):
    k = jnp.arange(max_length, dtype=jnp.float32)[:, None]
    i = jnp.arange(embedding_dim // 2, dtype=jnp.float32)[None, :]
    denominator = jnp.power(float(n), 2.0 * i / embedding_dim)
    P = jnp.zeros((max_length, embedding_dim), dtype=jnp.float32)
    P = P.at[:, 0::2].set(jnp.sin(k / denominator))
    P = P.at[:, 1::2].set(jnp.cos(k / denominator))
    return P


@functools.partial(jax.jit, static_argnames=("n",))
def _sc_lookup(xf, pf, table, penc, n):
    per_w = n // NW
    nsteps = per_w // CH
    ngroups = nsteps // NBUF
    mesh = plsc.VectorSubcoreMesh(core_axis_name="c", subcore_axis_name="s")

    @functools.partial(
        pl.kernel,
        mesh=mesh,
        out_type=jax.ShapeDtypeStruct((n, EMB), jnp.float32),
        scratch_types=[
            pltpu.VMEM((per_w,), jnp.int32),
            pltpu.VMEM((per_w,), jnp.int32),
            pltpu.VMEM((NBUF, CH, EMB), jnp.float32),
            pltpu.VMEM((MAXLEN, EMB), jnp.float32),
            pltpu.SemaphoreType.DMA((NBUF,)),
            pltpu.SemaphoreType.DMA((NBUF,)),
        ],
        compiler_params=pltpu.CompilerParams(
            use_tc_tiling_on_sc=False, needs_layout_passes=False),
    )
    def k(x_hbm, p_hbm, table_hbm, penc_hbm, out_hbm,
          xi_v, pi_v, rows_v, p_v, sem_t, sem_s):
        cid = lax.axis_index("c")
        sid = lax.axis_index("s")
        wid = sid * NC + cid
        w_base = wid * per_w

        pltpu.sync_copy(penc_hbm, p_v)
        pltpu.sync_copy(x_hbm.at[pl.ds(w_base, per_w)], xi_v)
        pltpu.sync_copy(p_hbm.at[pl.ds(w_base, per_w)], pi_v)

        lane = lax.iota(jnp.int32, 16)

        def issue_gather(g, b):
            xi = xi_v.at[pl.ds(g * CH, CH)]
            pltpu.async_copy(table_hbm.at[xi], rows_v.at[b], sem_t.at[b])

        def wait_gather(g, b):
            xi = xi_v.at[pl.ds(g * CH, CH)]
            pltpu.make_async_copy(table_hbm.at[xi], rows_v.at[b], sem_t.at[b]).wait()

        def wait_store(b):
            pltpu.make_async_copy(
                rows_v.at[b], out_hbm.at[pl.ds(w_base, CH)], sem_s.at[b]).wait()

        for b in range(NBUF - 1):
            issue_gather(b, b)

        def group(gr, carry):
            for b in range(NBUF):
                g = gr * NBUF + b
                pb = (b + NBUF - 1) % NBUF
                pg = g + NBUF - 1
                if b == 0:
                    @pl.when(gr >= 1)
                    def _():
                        wait_store(pb)

                    issue_gather(pg, pb)
                else:
                    @pl.when(pg < nsteps)
                    def _():
                        wait_store(pb)
                        issue_gather(pg, pb)

                wait_gather(g, b)

                def addrows(rr, c):
                    pos16 = pi_v[pl.ds(g * CH + rr * LANES, LANES)]
                    for j in range(LANES):
                        r = rr * LANES + j
                        prow = jnp.full((LANES,), pos16[j], dtype=jnp.int32)
                        for e in range(EMB // LANES):
                            sl = pl.ds(e * LANES, LANES)
                            pe = plsc.load_gather(p_v, [prow, lane + (e * LANES)])
                            rows_v[b, r, sl] = rows_v[b, r, sl] + pe
                    return c

                lax.fori_loop(0, CH // LANES, addrows, 0)
                pltpu.async_copy(
                    rows_v.at[b], out_hbm.at[pl.ds(w_base + g * CH, CH)],
                    sem_s.at[b])
            return carry

        lax.fori_loop(0, ngroups, group, 0)

        for j in range(NBUF):
            wait_store((nsteps - NBUF + j) % NBUF)

    return k(xf, pf, table, penc)


def kernel(x, pos, table):
    B, L = x.shape
    n = B * L
    xf = x.reshape(n).astype(jnp.int32)
    pf = pos.reshape(n).astype(jnp.int32)
    penc = _sincos_position_encoding(MAXLEN, EMB)
    out = _sc_lookup(xf, pf, table, penc, n)
    return out.reshape(B, L, EMB)
